# baseline (speedup 1.0000x reference)
import jax
import jax.numpy as jnp
from jax.experimental import pallas as pl

_B, _N, _S, _D1, _D2 = 8, 4096, 1024, 256, 512
_FI = 128
_C0, _C1 = 256, 256
_L = float(_B * _N)
_EPS = 1e-5
_NBA = 1024
_NB = 2048


def _ka(xyz1_ref, xyz2_ref, sq1_ref, sq2_ref, p2h_ref, p2l_ref, p1_ref,
        wg_ref, wx_ref, it_ref, g1_ref, x1_ref, st_ref):
    pb = pl.program_id(0)
    pn = pl.program_id(1)
    x1 = xyz1_ref[0]
    x2 = xyz2_ref[0]
    prod = jax.lax.dot_general(
        x1.astype(jnp.bfloat16), x2.astype(jnp.bfloat16),
        (((0,), (0,)), ((), ())),
        preferred_element_type=jnp.float32)
    sq1 = sq1_ref[0]
    sq2 = sq2_ref[0]
    d = (-2.0 * prod + sq1) + sq2

    iota = jax.lax.broadcasted_iota(jnp.int32, (_NBA, _S), 1)
    big = jnp.float32(jnp.inf)
    m1 = jnp.min(d, axis=1, keepdims=True)
    i1 = jnp.min(jnp.where(d == m1, iota, _S), axis=1, keepdims=True)
    d2 = jnp.where(iota == i1, big, d)
    m2 = jnp.min(d2, axis=1, keepdims=True)
    i2 = jnp.min(jnp.where(d2 == m2, iota, _S), axis=1, keepdims=True)
    d3 = jnp.where(iota == i2, big, d2)
    m3 = jnp.min(d3, axis=1, keepdims=True)
    i3 = jnp.min(jnp.where(d3 == m3, iota, _S), axis=1, keepdims=True)

    r1 = 1.0 / (m1 + 1e-8)
    r2 = 1.0 / (m2 + 1e-8)
    r3 = 1.0 / (m3 + 1e-8)
    norm = r1 + r2 + r3
    w1 = r1 / norm
    w2 = r2 / norm
    w3 = r3 / norm

    zero = jnp.float32(0.0)
    W = (jnp.where(iota == i1, w1, zero)
         + jnp.where(iota == i2, w2, zero)
         + jnp.where(iota == i3, w3, zero))

    bf = jnp.bfloat16
    f32_ = jnp.float32
    dims = (((1,), (1,)), ((), ()))
    Wh = W.astype(bf)
    Wl = (W - Wh.astype(f32_)).astype(bf)
    p2h = p2h_ref[0]
    it = (jax.lax.dot_general(p2h, Wh, dims, preferred_element_type=f32_)
          + jax.lax.dot_general(p2h, Wl, dims, preferred_element_type=f32_)
          + jax.lax.dot_general(p2l_ref[0], Wh, dims,
                                preferred_element_type=f32_)
          )
    itb = it.astype(bf)
    it_ref[0] = itb

    g1 = jnp.dot(wg_ref[...].astype(bf), itb,
                 preferred_element_type=jnp.float32)
    x1r = jnp.dot(wx_ref[...].astype(bf), p1_ref[0].astype(bf),
                  preferred_element_type=jnp.float32)
    g1_ref[0] = g1
    x1_ref[0] = x1r

    sg = jnp.sum(g1, axis=1, keepdims=True)
    qg = jnp.sum(g1 * g1, axis=1, keepdims=True)
    sx = jnp.sum(x1r, axis=1, keepdims=True)
    qx = jnp.sum(x1r * x1r, axis=1, keepdims=True)
    blk = jnp.concatenate([sg, qg, sx, qx], axis=1)

    @pl.when(jnp.logical_and(pb == 0, pn == 0))
    def _init():
        st_ref[...] = blk

    @pl.when(jnp.logical_or(pb != 0, pn != 0))
    def _acc():
        st_ref[...] = st_ref[...] + blk


def _kb(g1_ref, x1_ref, st_ref, bn4_ref, psiw_ref, pl_ref, ps_ref):
    pb = pl.program_id(0)
    pn = pl.program_id(1)
    st = st_ref[...]
    mg = st[:, 0:1] / _L
    vg = st[:, 1:2] / _L - mg * mg
    mx = st[:, 2:3] / _L
    vx = st[:, 3:4] / _L - mx * mx
    gg = bn4_ref[:, 0:1]
    bg = bn4_ref[:, 1:2]
    gx = bn4_ref[:, 2:3]
    bx = bn4_ref[:, 3:4]
    sg = gg * jax.lax.rsqrt(vg + _EPS)
    sx = gx * jax.lax.rsqrt(vx + _EPS)
    g1 = g1_ref[0] * sg + (bg - sg * mg)
    x1 = x1_ref[0] * sx + (bx - sx * mx)
    p = g1 + x1
    p = jnp.where(p >= 0, p, 0.2 * p)
    bf = jnp.bfloat16
    lin = jnp.dot(psiw_ref[...].astype(bf), p.astype(bf),
                  preferred_element_type=jnp.float32)
    pl_ref[0] = lin

    s1 = jnp.sum(lin).reshape(1, 1)
    s2 = jnp.sum(lin * lin).reshape(1, 1)
    row = jnp.concatenate([s1, s2, jnp.zeros((1, 126), jnp.float32)], axis=1)

    @pl.when(jnp.logical_and(pb == 0, pn == 0))
    def _init():
        ps_ref[...] = row

    @pl.when(jnp.logical_or(pb != 0, pn != 0))
    def _acc():
        ps_ref[...] = ps_ref[...] + row


def _kc(p1_ref, it_ref, pl_ref, ps_ref, psip_ref, w0a_ref, w0b_ref, b0_ref,
        c0_ref, st_ref):
    pb = pl.program_id(0)
    pn = pl.program_id(1)
    pm = ps_ref[0:1, 0:1] / _L
    pv = ps_ref[0:1, 1:2] / _L - pm * pm
    pgam = psip_ref[0:1, 1:2]
    pbet = psip_ref[0:1, 2:3]
    scale = pgam * jax.lax.rsqrt(pv + _EPS)
    lin = pl_ref[0]
    z = lin * scale + (pbet - scale * pm)
    psi = 1.0 / (1.0 + jnp.exp(-z))

    p1a = p1_ref[0] * psi
    bf = jnp.bfloat16
    f32 = jnp.float32
    c0 = (jnp.dot(w0a_ref[...].astype(bf), p1a.astype(bf),
                  preferred_element_type=f32)
          + jnp.dot(w0b_ref[...].astype(bf), it_ref[0],
                    preferred_element_type=f32)
          + b0_ref[...])
    c0_ref[0] = c0

    s = jnp.sum(c0, axis=1, keepdims=True)
    q = jnp.sum(c0 * c0, axis=1, keepdims=True)
    blk = jnp.concatenate([s, q], axis=1)

    @pl.when(jnp.logical_and(pb == 0, pn == 0))
    def _init():
        st_ref[...] = blk

    @pl.when(jnp.logical_or(pb != 0, pn != 0))
    def _acc():
        st_ref[...] = st_ref[...] + blk


def _kd(c0_ref, st_ref, bn0_ref, w1_ref, b1_ref, c1_ref, st1_ref):
    pb = pl.program_id(0)
    pn = pl.program_id(1)
    st = st_ref[...]
    m = st[:, 0:1] / _L
    v = st[:, 1:2] / _L - m * m
    g = bn0_ref[:, 0:1]
    b = bn0_ref[:, 1:2]
    sc = g * jax.lax.rsqrt(v + _EPS)
    h = c0_ref[0] * sc + (b - sc * m)
    h = jnp.where(h >= 0, h, 0.2 * h)
    bf = jnp.bfloat16
    c1 = jnp.dot(w1_ref[...].astype(bf), h.astype(bf),
                 preferred_element_type=jnp.float32) + b1_ref[...]
    c1_ref[0] = c1

    s = jnp.sum(c1, axis=1, keepdims=True)
    q = jnp.sum(c1 * c1, axis=1, keepdims=True)
    blk = jnp.concatenate([s, q], axis=1)

    @pl.when(jnp.logical_and(pb == 0, pn == 0))
    def _init():
        st1_ref[...] = blk

    @pl.when(jnp.logical_or(pb != 0, pn != 0))
    def _acc():
        st1_ref[...] = st1_ref[...] + blk


def _ke(c1_ref, st_ref, bn1_ref, o_ref):
    st = st_ref[...]
    m = st[:, 0:1] / _L
    v = st[:, 1:2] / _L - m * m
    g = bn1_ref[:, 0:1]
    b = bn1_ref[:, 1:2]
    sc = g * jax.lax.rsqrt(v + _EPS)
    h = c1_ref[0] * sc + (b - sc * m)
    o_ref[0] = jnp.where(h >= 0, h, 0.2 * h)


def _stage_a(xyz1p, xyz2p, sq1, sq2, p2h, p2l, points1, wg_w, wx_w):
    f32 = jnp.float32
    nga = _N // _NBA
    return pl.pallas_call(
        _ka,
        grid=(_B, nga),
        in_specs=[
            pl.BlockSpec((1, 8, _NBA), lambda b, n: (b, 0, n)),
            pl.BlockSpec((1, 8, _S), lambda b, n: (b, 0, 0)),
            pl.BlockSpec((1, _NBA, 1), lambda b, n: (b, n, 0)),
            pl.BlockSpec((1, 1, _S), lambda b, n: (b, 0, 0)),
            pl.BlockSpec((1, _D2, _S), lambda b, n: (b, 0, 0)),
            pl.BlockSpec((1, _D2, _S), lambda b, n: (b, 0, 0)),
            pl.BlockSpec((1, _D1, _NBA), lambda b, n: (b, 0, n)),
            pl.BlockSpec((_FI, _D2), lambda b, n: (0, 0)),
            pl.BlockSpec((_FI, _D1), lambda b, n: (0, 0)),
        ],
        out_specs=[
            pl.BlockSpec((1, _D2, _NBA), lambda b, n: (b, 0, n)),
            pl.BlockSpec((1, _FI, _NBA), lambda b, n: (b, 0, n)),
            pl.BlockSpec((1, _FI, _NBA), lambda b, n: (b, 0, n)),
            pl.BlockSpec((_FI, 4), lambda b, n: (0, 0)),
        ],
        out_shape=[
            jax.ShapeDtypeStruct((_B, _D2, _N), jnp.bfloat16),
            jax.ShapeDtypeStruct((_B, _FI, _N), f32),
            jax.ShapeDtypeStruct((_B, _FI, _N), f32),
            jax.ShapeDtypeStruct((_FI, 4), f32),
        ],
    )(xyz1p, xyz2p, sq1, sq2, p2h, p2l, points1, wg_w, wx_w)


def _stage_b(g1, x1, stA, bn4, psi_w):
    f32 = jnp.float32
    ngb = _N // _NB
    return pl.pallas_call(
        _kb,
        grid=(_B, ngb),
        in_specs=[
            pl.BlockSpec((1, _FI, _NB), lambda b, n: (b, 0, n)),
            pl.BlockSpec((1, _FI, _NB), lambda b, n: (b, 0, n)),
            pl.BlockSpec((_FI, 4), lambda b, n: (0, 0)),
            pl.BlockSpec((_FI, 4), lambda b, n: (0, 0)),
            pl.BlockSpec((1, _FI), lambda b, n: (0, 0)),
        ],
        out_specs=[
            pl.BlockSpec((1, 1, _NB), lambda b, n: (b, 0, n)),
            pl.BlockSpec((1, 128), lambda b, n: (0, 0)),
        ],
        out_shape=[
            jax.ShapeDtypeStruct((_B, 1, _N), f32),
            jax.ShapeDtypeStruct((1, 128), f32),
        ],
    )(g1, x1, stA, bn4, psi_w)


def _stage_c(points1, it, psilin, psist, psip, w0a, w0b, b0col):
    f32 = jnp.float32
    ngb = _N // _NB
    return pl.pallas_call(
        _kc,
        grid=(_B, ngb),
        in_specs=[
            pl.BlockSpec((1, _D1, _NB), lambda b, n: (b, 0, n)),
            pl.BlockSpec((1, _D2, _NB), lambda b, n: (b, 0, n)),
            pl.BlockSpec((1, 1, _NB), lambda b, n: (b, 0, n)),
            pl.BlockSpec((1, 128), lambda b, n: (0, 0)),
            pl.BlockSpec((1, 4), lambda b, n: (0, 0)),
            pl.BlockSpec((_C0, _D1), lambda b, n: (0, 0)),
            pl.BlockSpec((_C0, _D2), lambda b, n: (0, 0)),
            pl.BlockSpec((_C0, 1), lambda b, n: (0, 0)),
        ],
        out_specs=[
            pl.BlockSpec((1, _C0, _NB), lambda b, n: (b, 0, n)),
            pl.BlockSpec((_C0, 2), lambda b, n: (0, 0)),
        ],
        out_shape=[
            jax.ShapeDtypeStruct((_B, _C0, _N), f32),
            jax.ShapeDtypeStruct((_C0, 2), f32),
        ],
    )(points1, it, psilin, psist, psip, w0a, w0b, b0col)


def _stage_d(c0, st0, bn0, conv1_w, b1col):
    f32 = jnp.float32
    ngb = _N // _NB
    return pl.pallas_call(
        _kd,
        grid=(_B, ngb),
        in_specs=[
            pl.BlockSpec((1, _C0, _NB), lambda b, n: (b, 0, n)),
            pl.BlockSpec((_C0, 2), lambda b, n: (0, 0)),
            pl.BlockSpec((_C0, 2), lambda b, n: (0, 0)),
            pl.BlockSpec((_C1, _C0), lambda b, n: (0, 0)),
            pl.BlockSpec((_C1, 1), lambda b, n: (0, 0)),
        ],
        out_specs=[
            pl.BlockSpec((1, _C1, _NB), lambda b, n: (b, 0, n)),
            pl.BlockSpec((_C1, 2), lambda b, n: (0, 0)),
        ],
        out_shape=[
            jax.ShapeDtypeStruct((_B, _C1, _N), f32),
            jax.ShapeDtypeStruct((_C1, 2), f32),
        ],
    )(c0, st0, bn0, conv1_w, b1col)


def _stage_e(c1, st1, bn1):
    f32 = jnp.float32
    ngb = _N // _NB
    return pl.pallas_call(
        _ke,
        grid=(_B, ngb),
        in_specs=[
            pl.BlockSpec((1, _C1, _NB), lambda b, n: (b, 0, n)),
            pl.BlockSpec((_C1, 2), lambda b, n: (0, 0)),
            pl.BlockSpec((_C1, 2), lambda b, n: (0, 0)),
        ],
        out_specs=pl.BlockSpec((1, _C1, _NB), lambda b, n: (b, 0, n)),
        out_shape=jax.ShapeDtypeStruct((_B, _C1, _N), f32),
    )(c1, st1, bn1)


def kernel(xyz1, xyz2, points1, points2,
           wg_w, wg_b, wg_gamma, wg_beta,
           wx_w, wx_b, wx_gamma, wx_beta,
           psi_w, psi_b, psi_gamma, psi_beta,
           conv0_w, conv0_b, bn0_g, bn0_b,
           conv1_w, conv1_b, bn1_g, bn1_b):
    f32 = jnp.float32
    xyz1p = jnp.pad(xyz1, ((0, 0), (0, 5), (0, 0)))
    xyz2p = jnp.pad(xyz2, ((0, 0), (0, 5), (0, 0)))
    x1t = jnp.transpose(xyz1, (0, 2, 1))
    x2t = jnp.transpose(xyz2, (0, 2, 1))
    sq1 = jnp.sum(x1t ** 2, -1)[:, :, None]
    sq2 = jnp.sum(x2t ** 2, -1)[:, None, :]

    p2h = points2.astype(jnp.bfloat16)
    p2l = (points2 - p2h.astype(f32)).astype(jnp.bfloat16)

    it, g1, x1, stA = _stage_a(xyz1p, xyz2p, sq1, sq2, p2h, p2l, points1,
                               wg_w, wx_w)

    bn4 = jnp.concatenate([wg_gamma[:, None], wg_beta[:, None],
                           wx_gamma[:, None], wx_beta[:, None]], axis=1)
    psilin, psist = _stage_b(g1, x1, stA, bn4, psi_w)

    psip = jnp.concatenate([psi_b[:, None], psi_gamma[:, None],
                            psi_beta[:, None], jnp.zeros((1, 1), f32)],
                           axis=1)
    c0, st0 = _stage_c(points1, it, psilin, psist, psip,
                       conv0_w[:, :_D1], conv0_w[:, _D1:], conv0_b[:, None])

    bn0 = jnp.concatenate([bn0_g[:, None], bn0_b[:, None]], axis=1)
    c1, st1 = _stage_d(c0, st0, bn0, conv1_w, conv1_b[:, None])

    bn1 = jnp.concatenate([bn1_g[:, None], bn1_b[:, None]], axis=1)
    return _stage_e(c1, st1, bn1)


# speedup vs baseline: 1.0705x; 1.0705x over previous
import jax
import jax.numpy as jnp
from jax.experimental import pallas as pl

_B, _N, _S, _D1, _D2 = 8, 4096, 1024, 256, 512
_FI = 128
_C0, _C1 = 256, 256
_L = float(_B * _N)
_EPS = 1e-5
_NBA = 1024
_NB = 4096


def _ka(xyz1_ref, xyz2_ref, sq1_ref, sq2_ref, p2_ref, p1_ref,
        wg_ref, wx_ref, it_ref, g1_ref, x1_ref, st_ref):
    pb = pl.program_id(0)
    pn = pl.program_id(1)
    x1 = xyz1_ref[0]
    x2 = xyz2_ref[0]
    prod = jax.lax.dot_general(
        x1.astype(jnp.bfloat16), x2.astype(jnp.bfloat16),
        (((0,), (0,)), ((), ())),
        preferred_element_type=jnp.float32)
    sq1 = sq1_ref[0]
    sq2 = sq2_ref[0]
    d = (-2.0 * prod + sq1) + sq2

    iota = jax.lax.broadcasted_iota(jnp.int32, (_NBA, _S), 1)
    big = jnp.float32(jnp.inf)
    sf = _S
    m1 = jnp.min(d, axis=1, keepdims=True)
    i1 = jnp.min(jnp.where(d == m1, iota, sf), axis=1, keepdims=True)
    d2 = jnp.where(iota == i1, big, d)
    m2 = jnp.min(d2, axis=1, keepdims=True)
    i2 = jnp.min(jnp.where(d2 == m2, iota, sf), axis=1, keepdims=True)
    d3 = jnp.where(iota == i2, big, d2)
    m3 = jnp.min(d3, axis=1, keepdims=True)
    i3 = jnp.min(jnp.where(d3 == m3, iota, sf), axis=1, keepdims=True)

    r1 = 1.0 / (m1 + 1e-8)
    r2 = 1.0 / (m2 + 1e-8)
    r3 = 1.0 / (m3 + 1e-8)
    norm = r1 + r2 + r3
    w1 = r1 / norm
    w2 = r2 / norm
    w3 = r3 / norm

    zero = jnp.float32(0.0)
    W = (jnp.where(iota == i1, w1, zero)
         + jnp.where(iota == i2, w2, zero)
         + jnp.where(iota == i3, w3, zero))

    bf = jnp.bfloat16
    f32_ = jnp.float32
    dims = (((1,), (1,)), ((), ()))
    p2 = p2_ref[0]
    p2h = p2.astype(bf)
    p2l = (p2 - p2h.astype(f32_)).astype(bf)
    Wh = W.astype(bf)
    Wl = (W - Wh.astype(f32_)).astype(bf)
    it = (jax.lax.dot_general(p2h, Wh, dims, preferred_element_type=f32_)
          + jax.lax.dot_general(p2h, Wl, dims, preferred_element_type=f32_)
          + jax.lax.dot_general(p2l, Wh, dims,
                                preferred_element_type=f32_)
          )
    itb = it.astype(bf)
    it_ref[0] = itb

    g1 = jnp.dot(wg_ref[...].astype(bf), itb,
                 preferred_element_type=jnp.float32)
    x1r = jnp.dot(wx_ref[...].astype(bf), p1_ref[0].astype(bf),
                  preferred_element_type=jnp.float32)
    g1_ref[0] = g1
    x1_ref[0] = x1r

    sg = jnp.sum(g1, axis=1, keepdims=True)
    qg = jnp.sum(g1 * g1, axis=1, keepdims=True)
    sx = jnp.sum(x1r, axis=1, keepdims=True)
    qx = jnp.sum(x1r * x1r, axis=1, keepdims=True)
    blk = jnp.concatenate([sg, qg, sx, qx], axis=1)

    @pl.when(jnp.logical_and(pb == 0, pn == 0))
    def _init():
        st_ref[...] = blk

    @pl.when(jnp.logical_or(pb != 0, pn != 0))
    def _acc():
        st_ref[...] = st_ref[...] + blk


def _kb(g1_ref, x1_ref, st_ref, bn4_ref, psiw_ref, pl_ref, ps_ref):
    pb = pl.program_id(0)
    pn = pl.program_id(1)
    st = st_ref[...]
    mg = st[:, 0:1] / _L
    vg = st[:, 1:2] / _L - mg * mg
    mx = st[:, 2:3] / _L
    vx = st[:, 3:4] / _L - mx * mx
    gg = bn4_ref[:, 0:1]
    bg = bn4_ref[:, 1:2]
    gx = bn4_ref[:, 2:3]
    bx = bn4_ref[:, 3:4]
    sg = gg * jax.lax.rsqrt(vg + _EPS)
    sx = gx * jax.lax.rsqrt(vx + _EPS)
    g1 = g1_ref[0] * sg + (bg - sg * mg)
    x1 = x1_ref[0] * sx + (bx - sx * mx)
    p = g1 + x1
    p = jnp.where(p >= 0, p, 0.2 * p)
    bf = jnp.bfloat16
    lin = jnp.dot(psiw_ref[...].astype(bf), p.astype(bf),
                  preferred_element_type=jnp.float32)
    pl_ref[0] = lin

    s1 = jnp.sum(lin).reshape(1, 1)
    s2 = jnp.sum(lin * lin).reshape(1, 1)
    row = jnp.concatenate([s1, s2, jnp.zeros((1, 126), jnp.float32)], axis=1)

    @pl.when(jnp.logical_and(pb == 0, pn == 0))
    def _init():
        ps_ref[...] = row

    @pl.when(jnp.logical_or(pb != 0, pn != 0))
    def _acc():
        ps_ref[...] = ps_ref[...] + row


def _kc(p1_ref, it_ref, pl_ref, ps_ref, psip_ref, w0a_ref, w0b_ref, b0_ref,
        c0_ref, st_ref):
    pb = pl.program_id(0)
    pn = pl.program_id(1)
    pm = ps_ref[0:1, 0:1] / _L
    pv = ps_ref[0:1, 1:2] / _L - pm * pm
    pgam = psip_ref[0:1, 1:2]
    pbet = psip_ref[0:1, 2:3]
    scale = pgam * jax.lax.rsqrt(pv + _EPS)
    lin = pl_ref[0]
    z = lin * scale + (pbet - scale * pm)
    psi = 1.0 / (1.0 + jnp.exp(-z))

    p1a = p1_ref[0] * psi
    bf = jnp.bfloat16
    f32 = jnp.float32
    c0 = (jnp.dot(w0a_ref[...].astype(bf), p1a.astype(bf),
                  preferred_element_type=f32)
          + jnp.dot(w0b_ref[...].astype(bf), it_ref[0],
                    preferred_element_type=f32)
          + b0_ref[...])
    c0_ref[0] = c0

    s = jnp.sum(c0, axis=1, keepdims=True)
    q = jnp.sum(c0 * c0, axis=1, keepdims=True)
    blk = jnp.concatenate([s, q], axis=1)

    @pl.when(jnp.logical_and(pb == 0, pn == 0))
    def _init():
        st_ref[...] = blk

    @pl.when(jnp.logical_or(pb != 0, pn != 0))
    def _acc():
        st_ref[...] = st_ref[...] + blk


def _kd(c0_ref, st_ref, bn0_ref, w1_ref, b1_ref, c1_ref, st1_ref):
    pb = pl.program_id(0)
    pn = pl.program_id(1)
    st = st_ref[...]
    m = st[:, 0:1] / _L
    v = st[:, 1:2] / _L - m * m
    g = bn0_ref[:, 0:1]
    b = bn0_ref[:, 1:2]
    sc = g * jax.lax.rsqrt(v + _EPS)
    h = c0_ref[0] * sc + (b - sc * m)
    h = jnp.where(h >= 0, h, 0.2 * h)
    bf = jnp.bfloat16
    c1 = jnp.dot(w1_ref[...].astype(bf), h.astype(bf),
                 preferred_element_type=jnp.float32) + b1_ref[...]
    c1_ref[0] = c1

    s = jnp.sum(c1, axis=1, keepdims=True)
    q = jnp.sum(c1 * c1, axis=1, keepdims=True)
    blk = jnp.concatenate([s, q], axis=1)

    @pl.when(jnp.logical_and(pb == 0, pn == 0))
    def _init():
        st1_ref[...] = blk

    @pl.when(jnp.logical_or(pb != 0, pn != 0))
    def _acc():
        st1_ref[...] = st1_ref[...] + blk


def _ke(c1_ref, st_ref, bn1_ref, o_ref):
    st = st_ref[...]
    m = st[:, 0:1] / _L
    v = st[:, 1:2] / _L - m * m
    g = bn1_ref[:, 0:1]
    b = bn1_ref[:, 1:2]
    sc = g * jax.lax.rsqrt(v + _EPS)
    h = c1_ref[0] * sc + (b - sc * m)
    o_ref[0] = jnp.where(h >= 0, h, 0.2 * h)


def _stage_a(xyz1p, xyz2p, sq1, sq2, points2, points1, wg_w, wx_w):
    f32 = jnp.float32
    nga = _N // _NBA
    return pl.pallas_call(
        _ka,
        grid=(_B, nga),
        in_specs=[
            pl.BlockSpec((1, 8, _NBA), lambda b, n: (b, 0, n)),
            pl.BlockSpec((1, 8, _S), lambda b, n: (b, 0, 0)),
            pl.BlockSpec((1, _NBA, 1), lambda b, n: (b, n, 0)),
            pl.BlockSpec((1, 1, _S), lambda b, n: (b, 0, 0)),
            pl.BlockSpec((1, _D2, _S), lambda b, n: (b, 0, 0)),
            pl.BlockSpec((1, _D1, _NBA), lambda b, n: (b, 0, n)),
            pl.BlockSpec((_FI, _D2), lambda b, n: (0, 0)),
            pl.BlockSpec((_FI, _D1), lambda b, n: (0, 0)),
        ],
        out_specs=[
            pl.BlockSpec((1, _D2, _NBA), lambda b, n: (b, 0, n)),
            pl.BlockSpec((1, _FI, _NBA), lambda b, n: (b, 0, n)),
            pl.BlockSpec((1, _FI, _NBA), lambda b, n: (b, 0, n)),
            pl.BlockSpec((_FI, 4), lambda b, n: (0, 0)),
        ],
        out_shape=[
            jax.ShapeDtypeStruct((_B, _D2, _N), jnp.bfloat16),
            jax.ShapeDtypeStruct((_B, _FI, _N), f32),
            jax.ShapeDtypeStruct((_B, _FI, _N), f32),
            jax.ShapeDtypeStruct((_FI, 4), f32),
        ],
    )(xyz1p, xyz2p, sq1, sq2, points2, points1, wg_w, wx_w)


def _stage_b(g1, x1, stA, bn4, psi_w):
    f32 = jnp.float32
    ngb = _N // _NB
    return pl.pallas_call(
        _kb,
        grid=(_B, ngb),
        in_specs=[
            pl.BlockSpec((1, _FI, _NB), lambda b, n: (b, 0, n)),
            pl.BlockSpec((1, _FI, _NB), lambda b, n: (b, 0, n)),
            pl.BlockSpec((_FI, 4), lambda b, n: (0, 0)),
            pl.BlockSpec((_FI, 4), lambda b, n: (0, 0)),
            pl.BlockSpec((1, _FI), lambda b, n: (0, 0)),
        ],
        out_specs=[
            pl.BlockSpec((1, 1, _NB), lambda b, n: (b, 0, n)),
            pl.BlockSpec((1, 128), lambda b, n: (0, 0)),
        ],
        out_shape=[
            jax.ShapeDtypeStruct((_B, 1, _N), f32),
            jax.ShapeDtypeStruct((1, 128), f32),
        ],
    )(g1, x1, stA, bn4, psi_w)


def _stage_c(points1, it, psilin, psist, psip, w0a, w0b, b0col):
    f32 = jnp.float32
    ngb = _N // _NB
    return pl.pallas_call(
        _kc,
        grid=(_B, ngb),
        in_specs=[
            pl.BlockSpec((1, _D1, _NB), lambda b, n: (b, 0, n)),
            pl.BlockSpec((1, _D2, _NB), lambda b, n: (b, 0, n)),
            pl.BlockSpec((1, 1, _NB), lambda b, n: (b, 0, n)),
            pl.BlockSpec((1, 128), lambda b, n: (0, 0)),
            pl.BlockSpec((1, 4), lambda b, n: (0, 0)),
            pl.BlockSpec((_C0, _D1), lambda b, n: (0, 0)),
            pl.BlockSpec((_C0, _D2), lambda b, n: (0, 0)),
            pl.BlockSpec((_C0, 1), lambda b, n: (0, 0)),
        ],
        out_specs=[
            pl.BlockSpec((1, _C0, _NB), lambda b, n: (b, 0, n)),
            pl.BlockSpec((_C0, 2), lambda b, n: (0, 0)),
        ],
        out_shape=[
            jax.ShapeDtypeStruct((_B, _C0, _N), f32),
            jax.ShapeDtypeStruct((_C0, 2), f32),
        ],
    )(points1, it, psilin, psist, psip, w0a, w0b, b0col)


def _stage_d(c0, st0, bn0, conv1_w, b1col):
    f32 = jnp.float32
    ngb = _N // _NB
    return pl.pallas_call(
        _kd,
        grid=(_B, ngb),
        in_specs=[
            pl.BlockSpec((1, _C0, _NB), lambda b, n: (b, 0, n)),
            pl.BlockSpec((_C0, 2), lambda b, n: (0, 0)),
            pl.BlockSpec((_C0, 2), lambda b, n: (0, 0)),
            pl.BlockSpec((_C1, _C0), lambda b, n: (0, 0)),
            pl.BlockSpec((_C1, 1), lambda b, n: (0, 0)),
        ],
        out_specs=[
            pl.BlockSpec((1, _C1, _NB), lambda b, n: (b, 0, n)),
            pl.BlockSpec((_C1, 2), lambda b, n: (0, 0)),
        ],
        out_shape=[
            jax.ShapeDtypeStruct((_B, _C1, _N), f32),
            jax.ShapeDtypeStruct((_C1, 2), f32),
        ],
    )(c0, st0, bn0, conv1_w, b1col)


def _stage_e(c1, st1, bn1):
    f32 = jnp.float32
    ngb = _N // _NB
    return pl.pallas_call(
        _ke,
        grid=(_B, ngb),
        in_specs=[
            pl.BlockSpec((1, _C1, _NB), lambda b, n: (b, 0, n)),
            pl.BlockSpec((_C1, 2), lambda b, n: (0, 0)),
            pl.BlockSpec((_C1, 2), lambda b, n: (0, 0)),
        ],
        out_specs=pl.BlockSpec((1, _C1, _NB), lambda b, n: (b, 0, n)),
        out_shape=jax.ShapeDtypeStruct((_B, _C1, _N), f32),
    )(c1, st1, bn1)


def kernel(xyz1, xyz2, points1, points2,
           wg_w, wg_b, wg_gamma, wg_beta,
           wx_w, wx_b, wx_gamma, wx_beta,
           psi_w, psi_b, psi_gamma, psi_beta,
           conv0_w, conv0_b, bn0_g, bn0_b,
           conv1_w, conv1_b, bn1_g, bn1_b):
    f32 = jnp.float32
    xyz1p = jnp.pad(xyz1, ((0, 0), (0, 5), (0, 0)))
    xyz2p = jnp.pad(xyz2, ((0, 0), (0, 5), (0, 0)))
    x1t = jnp.transpose(xyz1, (0, 2, 1))
    x2t = jnp.transpose(xyz2, (0, 2, 1))
    sq1 = jnp.sum(x1t ** 2, -1)[:, :, None]
    sq2 = jnp.sum(x2t ** 2, -1)[:, None, :]

    it, g1, x1, stA = _stage_a(xyz1p, xyz2p, sq1, sq2, points2, points1,
                               wg_w, wx_w)

    bn4 = jnp.concatenate([wg_gamma[:, None], wg_beta[:, None],
                           wx_gamma[:, None], wx_beta[:, None]], axis=1)
    psilin, psist = _stage_b(g1, x1, stA, bn4, psi_w)

    psip = jnp.concatenate([psi_b[:, None], psi_gamma[:, None],
                            psi_beta[:, None], jnp.zeros((1, 1), f32)],
                           axis=1)
    c0, st0 = _stage_c(points1, it, psilin, psist, psip,
                       conv0_w[:, :_D1], conv0_w[:, _D1:], conv0_b[:, None])

    bn0 = jnp.concatenate([bn0_g[:, None], bn0_b[:, None]], axis=1)
    c1, st1 = _stage_d(c0, st0, bn0, conv1_w, conv1_b[:, None])

    bn1 = jnp.concatenate([bn1_g[:, None], bn1_b[:, None]], axis=1)
    return _stage_e(c1, st1, bn1)


# speedup vs baseline: 1.1707x; 1.0937x over previous
import jax
import jax.numpy as jnp
from jax.experimental import pallas as pl

_B, _N, _S, _D1, _D2 = 8, 4096, 1024, 256, 512
_FI = 128
_C0, _C1 = 256, 256
_L = float(_B * _N)
_EPS = 1e-5
_NBA = 1024
_NB = 4096


def _ka(xyz1_ref, xyz2_ref, sq1_ref, sq2_ref, p2_ref, p1_ref,
        wg_ref, wx_ref, it_ref, g1_ref, x1_ref, st_ref):
    pb = pl.program_id(0)
    pn = pl.program_id(1)
    x1 = xyz1_ref[0]
    x2 = xyz2_ref[0]
    prod = jax.lax.dot_general(
        x1.astype(jnp.bfloat16), x2.astype(jnp.bfloat16),
        (((0,), (0,)), ((), ())),
        preferred_element_type=jnp.float32)
    sq1 = sq1_ref[0]
    sq2 = sq2_ref[0]
    d = (-2.0 * prod + sq1) + sq2

    iota = jax.lax.broadcasted_iota(jnp.int32, (_NBA, _S), 1)
    big = jnp.float32(jnp.inf)
    sf = _S
    m1 = jnp.min(d, axis=1, keepdims=True)
    i1 = jnp.min(jnp.where(d == m1, iota, sf), axis=1, keepdims=True)
    d2 = jnp.where(iota == i1, big, d)
    m2 = jnp.min(d2, axis=1, keepdims=True)
    i2 = jnp.min(jnp.where(d2 == m2, iota, sf), axis=1, keepdims=True)
    d3 = jnp.where(iota == i2, big, d2)
    m3 = jnp.min(d3, axis=1, keepdims=True)
    i3 = jnp.min(jnp.where(d3 == m3, iota, sf), axis=1, keepdims=True)

    r1 = 1.0 / (m1 + 1e-8)
    r2 = 1.0 / (m2 + 1e-8)
    r3 = 1.0 / (m3 + 1e-8)
    norm = r1 + r2 + r3
    w1 = r1 / norm
    w2 = r2 / norm
    w3 = r3 / norm

    zero = jnp.float32(0.0)
    W = (jnp.where(iota == i1, w1, zero)
         + jnp.where(iota == i2, w2, zero)
         + jnp.where(iota == i3, w3, zero))

    bf = jnp.bfloat16
    f32_ = jnp.float32
    dims = (((1,), (1,)), ((), ()))
    p2h = p2_ref[0].astype(bf)
    Wh = W.astype(bf)
    Wl = (W - Wh.astype(f32_)).astype(bf)
    it = (jax.lax.dot_general(p2h, Wh, dims, preferred_element_type=f32_)
          + jax.lax.dot_general(p2h, Wl, dims, preferred_element_type=f32_)
          )
    itb = it.astype(bf)
    it_ref[0] = itb

    g1 = jnp.dot(wg_ref[...].astype(bf), itb,
                 preferred_element_type=jnp.float32)
    x1r = jnp.dot(wx_ref[...].astype(bf), p1_ref[0].astype(bf),
                  preferred_element_type=jnp.float32)
    g1_ref[0] = g1
    x1_ref[0] = x1r

    sg = jnp.sum(g1, axis=1, keepdims=True)
    qg = jnp.sum(g1 * g1, axis=1, keepdims=True)
    sx = jnp.sum(x1r, axis=1, keepdims=True)
    qx = jnp.sum(x1r * x1r, axis=1, keepdims=True)
    blk = jnp.concatenate([sg, qg, sx, qx], axis=1)

    @pl.when(jnp.logical_and(pb == 0, pn == 0))
    def _init():
        st_ref[...] = blk

    @pl.when(jnp.logical_or(pb != 0, pn != 0))
    def _acc():
        st_ref[...] = st_ref[...] + blk


def _kb(g1_ref, x1_ref, st_ref, bn4_ref, psiw_ref, pl_ref, ps_ref):
    pb = pl.program_id(0)
    pn = pl.program_id(1)
    st = st_ref[...]
    mg = st[:, 0:1] / _L
    vg = st[:, 1:2] / _L - mg * mg
    mx = st[:, 2:3] / _L
    vx = st[:, 3:4] / _L - mx * mx
    gg = bn4_ref[:, 0:1]
    bg = bn4_ref[:, 1:2]
    gx = bn4_ref[:, 2:3]
    bx = bn4_ref[:, 3:4]
    sg = gg * jax.lax.rsqrt(vg + _EPS)
    sx = gx * jax.lax.rsqrt(vx + _EPS)
    g1 = g1_ref[0] * sg + (bg - sg * mg)
    x1 = x1_ref[0] * sx + (bx - sx * mx)
    p = g1 + x1
    p = jnp.where(p >= 0, p, 0.2 * p)
    bf = jnp.bfloat16
    lin = jnp.dot(psiw_ref[...].astype(bf), p.astype(bf),
                  preferred_element_type=jnp.float32)
    pl_ref[0] = lin

    s1 = jnp.sum(lin).reshape(1, 1)
    s2 = jnp.sum(lin * lin).reshape(1, 1)
    row = jnp.concatenate([s1, s2, jnp.zeros((1, 126), jnp.float32)], axis=1)

    @pl.when(jnp.logical_and(pb == 0, pn == 0))
    def _init():
        ps_ref[...] = row

    @pl.when(jnp.logical_or(pb != 0, pn != 0))
    def _acc():
        ps_ref[...] = ps_ref[...] + row


def _kc(p1_ref, it_ref, pl_ref, ps_ref, psip_ref, w0a_ref, w0b_ref, b0_ref,
        c0_ref, st_ref):
    pb = pl.program_id(0)
    pn = pl.program_id(1)
    pm = ps_ref[0:1, 0:1] / _L
    pv = ps_ref[0:1, 1:2] / _L - pm * pm
    pgam = psip_ref[0:1, 1:2]
    pbet = psip_ref[0:1, 2:3]
    scale = pgam * jax.lax.rsqrt(pv + _EPS)
    lin = pl_ref[0]
    z = lin * scale + (pbet - scale * pm)
    psi = 1.0 / (1.0 + jnp.exp(-z))

    p1a = p1_ref[0] * psi
    bf = jnp.bfloat16
    f32 = jnp.float32
    c0 = (jnp.dot(w0a_ref[...].astype(bf), p1a.astype(bf),
                  preferred_element_type=f32)
          + jnp.dot(w0b_ref[...].astype(bf), it_ref[0],
                    preferred_element_type=f32)
          + b0_ref[...])
    c0_ref[0] = c0

    s = jnp.sum(c0, axis=1, keepdims=True)
    q = jnp.sum(c0 * c0, axis=1, keepdims=True)
    blk = jnp.concatenate([s, q], axis=1)

    @pl.when(jnp.logical_and(pb == 0, pn == 0))
    def _init():
        st_ref[...] = blk

    @pl.when(jnp.logical_or(pb != 0, pn != 0))
    def _acc():
        st_ref[...] = st_ref[...] + blk


def _kd(c0_ref, st_ref, bn0_ref, w1_ref, b1_ref, c1_ref, st1_ref):
    pb = pl.program_id(0)
    pn = pl.program_id(1)
    st = st_ref[...]
    m = st[:, 0:1] / _L
    v = st[:, 1:2] / _L - m * m
    g = bn0_ref[:, 0:1]
    b = bn0_ref[:, 1:2]
    sc = g * jax.lax.rsqrt(v + _EPS)
    h = c0_ref[0] * sc + (b - sc * m)
    h = jnp.where(h >= 0, h, 0.2 * h)
    bf = jnp.bfloat16
    c1 = jnp.dot(w1_ref[...].astype(bf), h.astype(bf),
                 preferred_element_type=jnp.float32) + b1_ref[...]
    c1_ref[0] = c1

    s = jnp.sum(c1, axis=1, keepdims=True)
    q = jnp.sum(c1 * c1, axis=1, keepdims=True)
    blk = jnp.concatenate([s, q], axis=1)

    @pl.when(jnp.logical_and(pb == 0, pn == 0))
    def _init():
        st1_ref[...] = blk

    @pl.when(jnp.logical_or(pb != 0, pn != 0))
    def _acc():
        st1_ref[...] = st1_ref[...] + blk


def _ke(c1_ref, st_ref, bn1_ref, o_ref):
    st = st_ref[...]
    m = st[:, 0:1] / _L
    v = st[:, 1:2] / _L - m * m
    g = bn1_ref[:, 0:1]
    b = bn1_ref[:, 1:2]
    sc = g * jax.lax.rsqrt(v + _EPS)
    h = c1_ref[0] * sc + (b - sc * m)
    o_ref[0] = jnp.where(h >= 0, h, 0.2 * h)


def _stage_a(xyz1p, xyz2p, sq1, sq2, points2, points1, wg_w, wx_w):
    f32 = jnp.float32
    nga = _N // _NBA
    return pl.pallas_call(
        _ka,
        grid=(_B, nga),
        in_specs=[
            pl.BlockSpec((1, 8, _NBA), lambda b, n: (b, 0, n)),
            pl.BlockSpec((1, 8, _S), lambda b, n: (b, 0, 0)),
            pl.BlockSpec((1, _NBA, 1), lambda b, n: (b, n, 0)),
            pl.BlockSpec((1, 1, _S), lambda b, n: (b, 0, 0)),
            pl.BlockSpec((1, _D2, _S), lambda b, n: (b, 0, 0)),
            pl.BlockSpec((1, _D1, _NBA), lambda b, n: (b, 0, n)),
            pl.BlockSpec((_FI, _D2), lambda b, n: (0, 0)),
            pl.BlockSpec((_FI, _D1), lambda b, n: (0, 0)),
        ],
        out_specs=[
            pl.BlockSpec((1, _D2, _NBA), lambda b, n: (b, 0, n)),
            pl.BlockSpec((1, _FI, _NBA), lambda b, n: (b, 0, n)),
            pl.BlockSpec((1, _FI, _NBA), lambda b, n: (b, 0, n)),
            pl.BlockSpec((_FI, 4), lambda b, n: (0, 0)),
        ],
        out_shape=[
            jax.ShapeDtypeStruct((_B, _D2, _N), jnp.bfloat16),
            jax.ShapeDtypeStruct((_B, _FI, _N), f32),
            jax.ShapeDtypeStruct((_B, _FI, _N), f32),
            jax.ShapeDtypeStruct((_FI, 4), f32),
        ],
    )(xyz1p, xyz2p, sq1, sq2, points2, points1, wg_w, wx_w)


def _stage_b(g1, x1, stA, bn4, psi_w):
    f32 = jnp.float32
    ngb = _N // _NB
    return pl.pallas_call(
        _kb,
        grid=(_B, ngb),
        in_specs=[
            pl.BlockSpec((1, _FI, _NB), lambda b, n: (b, 0, n)),
            pl.BlockSpec((1, _FI, _NB), lambda b, n: (b, 0, n)),
            pl.BlockSpec((_FI, 4), lambda b, n: (0, 0)),
            pl.BlockSpec((_FI, 4), lambda b, n: (0, 0)),
            pl.BlockSpec((1, _FI), lambda b, n: (0, 0)),
        ],
        out_specs=[
            pl.BlockSpec((1, 1, _NB), lambda b, n: (b, 0, n)),
            pl.BlockSpec((1, 128), lambda b, n: (0, 0)),
        ],
        out_shape=[
            jax.ShapeDtypeStruct((_B, 1, _N), f32),
            jax.ShapeDtypeStruct((1, 128), f32),
        ],
    )(g1, x1, stA, bn4, psi_w)


def _stage_c(points1, it, psilin, psist, psip, w0a, w0b, b0col):
    f32 = jnp.float32
    ngb = _N // _NB
    return pl.pallas_call(
        _kc,
        grid=(_B, ngb),
        in_specs=[
            pl.BlockSpec((1, _D1, _NB), lambda b, n: (b, 0, n)),
            pl.BlockSpec((1, _D2, _NB), lambda b, n: (b, 0, n)),
            pl.BlockSpec((1, 1, _NB), lambda b, n: (b, 0, n)),
            pl.BlockSpec((1, 128), lambda b, n: (0, 0)),
            pl.BlockSpec((1, 4), lambda b, n: (0, 0)),
            pl.BlockSpec((_C0, _D1), lambda b, n: (0, 0)),
            pl.BlockSpec((_C0, _D2), lambda b, n: (0, 0)),
            pl.BlockSpec((_C0, 1), lambda b, n: (0, 0)),
        ],
        out_specs=[
            pl.BlockSpec((1, _C0, _NB), lambda b, n: (b, 0, n)),
            pl.BlockSpec((_C0, 2), lambda b, n: (0, 0)),
        ],
        out_shape=[
            jax.ShapeDtypeStruct((_B, _C0, _N), f32),
            jax.ShapeDtypeStruct((_C0, 2), f32),
        ],
    )(points1, it, psilin, psist, psip, w0a, w0b, b0col)


def _stage_d(c0, st0, bn0, conv1_w, b1col):
    f32 = jnp.float32
    ngb = _N // _NB
    return pl.pallas_call(
        _kd,
        grid=(_B, ngb),
        in_specs=[
            pl.BlockSpec((1, _C0, _NB), lambda b, n: (b, 0, n)),
            pl.BlockSpec((_C0, 2), lambda b, n: (0, 0)),
            pl.BlockSpec((_C0, 2), lambda b, n: (0, 0)),
            pl.BlockSpec((_C1, _C0), lambda b, n: (0, 0)),
            pl.BlockSpec((_C1, 1), lambda b, n: (0, 0)),
        ],
        out_specs=[
            pl.BlockSpec((1, _C1, _NB), lambda b, n: (b, 0, n)),
            pl.BlockSpec((_C1, 2), lambda b, n: (0, 0)),
        ],
        out_shape=[
            jax.ShapeDtypeStruct((_B, _C1, _N), f32),
            jax.ShapeDtypeStruct((_C1, 2), f32),
        ],
    )(c0, st0, bn0, conv1_w, b1col)


def _stage_e(c1, st1, bn1):
    f32 = jnp.float32
    ngb = _N // _NB
    return pl.pallas_call(
        _ke,
        grid=(_B, ngb),
        in_specs=[
            pl.BlockSpec((1, _C1, _NB), lambda b, n: (b, 0, n)),
            pl.BlockSpec((_C1, 2), lambda b, n: (0, 0)),
            pl.BlockSpec((_C1, 2), lambda b, n: (0, 0)),
        ],
        out_specs=pl.BlockSpec((1, _C1, _NB), lambda b, n: (b, 0, n)),
        out_shape=jax.ShapeDtypeStruct((_B, _C1, _N), f32),
    )(c1, st1, bn1)


def kernel(xyz1, xyz2, points1, points2,
           wg_w, wg_b, wg_gamma, wg_beta,
           wx_w, wx_b, wx_gamma, wx_beta,
           psi_w, psi_b, psi_gamma, psi_beta,
           conv0_w, conv0_b, bn0_g, bn0_b,
           conv1_w, conv1_b, bn1_g, bn1_b):
    f32 = jnp.float32
    xyz1p = jnp.pad(xyz1, ((0, 0), (0, 5), (0, 0)))
    xyz2p = jnp.pad(xyz2, ((0, 0), (0, 5), (0, 0)))
    x1t = jnp.transpose(xyz1, (0, 2, 1))
    x2t = jnp.transpose(xyz2, (0, 2, 1))
    sq1 = jnp.sum(x1t ** 2, -1)[:, :, None]
    sq2 = jnp.sum(x2t ** 2, -1)[:, None, :]

    it, g1, x1, stA = _stage_a(xyz1p, xyz2p, sq1, sq2, points2, points1,
                               wg_w, wx_w)

    bn4 = jnp.concatenate([wg_gamma[:, None], wg_beta[:, None],
                           wx_gamma[:, None], wx_beta[:, None]], axis=1)
    psilin, psist = _stage_b(g1, x1, stA, bn4, psi_w)

    psip = jnp.concatenate([psi_b[:, None], psi_gamma[:, None],
                            psi_beta[:, None], jnp.zeros((1, 1), f32)],
                           axis=1)
    c0, st0 = _stage_c(points1, it, psilin, psist, psip,
                       conv0_w[:, :_D1], conv0_w[:, _D1:], conv0_b[:, None])

    bn0 = jnp.concatenate([bn0_g[:, None], bn0_b[:, None]], axis=1)
    c1, st1 = _stage_d(c0, st0, bn0, conv1_w, conv1_b[:, None])

    bn1 = jnp.concatenate([bn1_g[:, None], bn1_b[:, None]], axis=1)
    return _stage_e(c1, st1, bn1)
